# baseline (speedup 1.0000x reference)
import functools

import jax
import jax.numpy as jnp
from jax import lax
from jax.experimental import pallas as pl
from jax.experimental.pallas import tpu as pltpu
from jax.experimental.pallas import tpu_sc as plsc

_L = 16
_NC = 2
_BITS = 10
_BINS = 1 << _BITS
_KMAX = 0x3FFFFFFF


def _score_body(x_ref, w1_ref, b1_ref, w2_ref, out_ref):
    xb = x_ref[0]
    h = jnp.dot(w1_ref[...], xb, preferred_element_type=jnp.float32)
    h = jax.nn.relu(h + b1_ref[...][:, None])
    s = jnp.dot(w2_ref[...], h, preferred_element_type=jnp.float32)
    out_ref[0] = jax.nn.sigmoid(s)


def _scores(x, W1, b1, W2):
    B, C, N = x.shape
    NCH = 4
    out = pl.pallas_call(
        _score_body,
        grid=(B, NCH),
        in_specs=[
            pl.BlockSpec((1, C, N // NCH), lambda b, n: (b, 0, n)),
            pl.BlockSpec((C // 4, C), lambda b, n: (0, 0)),
            pl.BlockSpec((C // 4,), lambda b, n: (0,)),
            pl.BlockSpec((1, C // 4), lambda b, n: (0, 0)),
        ],
        out_specs=pl.BlockSpec((1, 1, N // NCH), lambda b, n: (b, 0, n)),
        out_shape=jax.ShapeDtypeStruct((B, 1, N), jnp.float32),
    )(x, W1, b1, W2)
    return out[:, 0, :]


def _sort_body(score_hbm, idx_hbm, sck_hbm,
               srow, val0, val1, hist0, hist1, hist2, hist3, tvec, sck_v):
    B, N = score_hbm.shape
    K = N // 2
    vpl = N // _L
    qv = vpl // 4
    wid = lax.axis_index("s") * _NC + lax.axis_index("c")

    @pl.when(wid < B)
    def _():
        lane = lax.iota(jnp.int32, _L)
        pltpu.sync_copy(score_hbm.at[wid], srow)

        lvpl = lane * vpl
        ones = jnp.ones((_L,), jnp.int32)
        zeros = jnp.zeros((_L,), jnp.int32)

        def key_at(vi, iv):
            if vi is None:
                vv = iv
            else:
                vv = plsc.load_gather(vi, [iv])
            u = plsc.bitcast(plsc.load_gather(srow, [vv]), jnp.int32)
            return _KMAX - u, vv

        def rs_pass(vi, vo, shift):
            @plsc.parallel_loop(0, _BINS, unroll=4)
            def _(i):
                sl = pl.ds(i * _L, _L)
                hist0[sl] = zeros
                hist1[sl] = zeros
                hist2[sl] = zeros
                hist3[sl] = zeros

            def h1(v, _):
                k0, _v0 = key_at(vi, lvpl + v)
                k1, _v1 = key_at(vi, lvpl + (qv + v))
                k2, _v2 = key_at(vi, lvpl + (2 * qv + v))
                k3, _v3 = key_at(vi, lvpl + (3 * qv + v))
                a0 = ((k0 >> shift) & (_BINS - 1)) * _L + lane
                a1 = ((k1 >> shift) & (_BINS - 1)) * _L + lane
                a2 = ((k2 >> shift) & (_BINS - 1)) * _L + lane
                a3 = ((k3 >> shift) & (_BINS - 1)) * _L + lane
                plsc.addupdate_scatter(hist0, [a0], ones)
                plsc.addupdate_scatter(hist1, [a1], ones)
                plsc.addupdate_scatter(hist2, [a2], ones)
                plsc.addupdate_scatter(hist3, [a3], ones)
                return 0

            lax.fori_loop(0, qv, h1, 0, unroll=2)

            @plsc.parallel_loop(0, _BINS, unroll=4)
            def _(i):
                sl = pl.ds(i * _L, _L)
                c0 = hist0[sl]
                c1 = hist1[sl]
                c2 = hist2[sl]
                c3 = hist3[sl]
                t = c0 + c1 + c2 + c3
                ic = plsc.cumsum(t)
                r0 = ic - t
                hist0[sl] = r0
                hist1[sl] = r0 + c0
                hist2[sl] = r0 + c0 + c1
                hist3[sl] = ic

            def lvl2(i, carry):
                addr15 = (i * _L + lane) * _L + (_L - 1)
                t = plsc.load_gather(hist3, [addr15])
                eb = (plsc.cumsum(t) - t) + carry
                plsc.store_scatter(tvec, [i * _L + lane], eb)
                return carry + jnp.sum(t)

            lax.fori_loop(0, _BINS // _L, lvl2, jnp.int32(0))

            @plsc.parallel_loop(0, _BINS, unroll=4)
            def _(i):
                sl = pl.ds(i * _L, _L)
                base = plsc.load_gather(tvec, [lane * 0 + i])
                hist0[sl] = hist0[sl] + base
                hist1[sl] = hist1[sl] + base
                hist2[sl] = hist2[sl] + base
                hist3[sl] = hist3[sl] + base

            def h3(v, _):
                k0, v0 = key_at(vi, lvpl + v)
                k1, v1 = key_at(vi, lvpl + (qv + v))
                k2, v2 = key_at(vi, lvpl + (2 * qv + v))
                k3, v3 = key_at(vi, lvpl + (3 * qv + (qv - 1 - v)))
                a0 = ((k0 >> shift) & (_BINS - 1)) * _L + lane
                a1 = ((k1 >> shift) & (_BINS - 1)) * _L + lane
                a2 = ((k2 >> shift) & (_BINS - 1)) * _L + lane
                a3 = ((k3 >> shift) & (_BINS - 1)) * _L + lane
                cur0 = plsc.load_gather(hist0, [a0])
                cur1 = plsc.load_gather(hist1, [a1])
                cur2 = plsc.load_gather(hist2, [a2])
                cur3 = plsc.load_gather(hist3, [a3]) - 1
                plsc.store_scatter(hist0, [a0], cur0 + 1)
                plsc.store_scatter(hist1, [a1], cur1 + 1)
                plsc.store_scatter(hist2, [a2], cur2 + 1)
                plsc.store_scatter(hist3, [a3], cur3)
                plsc.store_scatter(vo, [cur0], v0)
                plsc.store_scatter(vo, [cur1], v1)
                plsc.store_scatter(vo, [cur2], v2)
                plsc.store_scatter(vo, [cur3], v3)
                return 0

            lax.fori_loop(0, qv, h3, 0, unroll=2)

        rs_pass(None, val1, 0)
        rs_pass(val1, val0, _BITS)
        rs_pass(val0, val1, 2 * _BITS)

        @plsc.parallel_loop(0, K // _L, unroll=4)
        def _(i):
            sl = pl.ds(i * _L, _L)
            sck_v[sl] = plsc.load_gather(srow, [val1[sl]])

        pltpu.sync_copy(val1.at[pl.ds(0, K)], idx_hbm.at[wid])
        pltpu.sync_copy(sck_v, sck_hbm.at[wid])


def _sort(score):
    B, N = score.shape
    K = N // 2
    mesh = plsc.VectorSubcoreMesh(core_axis_name="c", subcore_axis_name="s")
    f = pl.kernel(
        _sort_body,
        out_type=[
            jax.ShapeDtypeStruct((B, K), jnp.int32),
            jax.ShapeDtypeStruct((B, K), jnp.float32),
        ],
        mesh=mesh,
        scratch_types=[
            pltpu.VMEM((N,), jnp.float32),
            pltpu.VMEM((N,), jnp.int32),
            pltpu.VMEM((N,), jnp.int32),
            pltpu.VMEM((_BINS * _L,), jnp.int32),
            pltpu.VMEM((_BINS * _L,), jnp.int32),
            pltpu.VMEM((_BINS * _L,), jnp.int32),
            pltpu.VMEM((_BINS * _L,), jnp.int32),
            pltpu.VMEM((_BINS,), jnp.int32),
            pltpu.VMEM((K,), jnp.float32),
        ],
        compiler_params=pltpu.CompilerParams(needs_layout_passes=False),
    )
    return f(score)


def _gather_body(pos_hbm, feat_hbm, x_hbm, idx_hbm, sck_hbm,
                 posk_hbm, lfk_hbm, fk_hbm,
                 idx_v, sck_v, rowa0, rowb0, rowa1, rowb1,
                 outa0, outb0, outa1, outb1,
                 isa0, isb0, isa1, isb1, osa0, osb0, osa1, osb1):
    B = pos_hbm.shape[0]
    C = x_hbm.shape[1]
    CF = feat_hbm.shape[1]
    N = x_hbm.shape[2]
    K = N // 2
    wid = lax.axis_index("s") * _NC + lax.axis_index("c")
    b = wid // 4
    sub = wid % 4
    rows_a = (rowa0, rowa1)
    rows_b = (rowb0, rowb1)
    outs_a = (outa0, outa1)
    outs_b = (outb0, outb1)
    isa = (isa0, isa1)
    isb = (isb0, isb1)
    osa = (osa0, osa1)
    osb = (osb0, osb1)

    pltpu.sync_copy(idx_hbm.at[b], idx_v)
    pltpu.sync_copy(sck_hbm.at[b], sck_v)

    def start_in(p, par):
        @pl.when(p < 16)
        def _():
            pltpu.async_copy(x_hbm.at[b, sub * 32 + 2 * p], rows_a[par], isa[par])
            pltpu.async_copy(x_hbm.at[b, sub * 32 + 2 * p + 1], rows_b[par],
                             isb[par])

        @pl.when(p >= 16)
        def _():
            pltpu.async_copy(feat_hbm.at[b, sub * 16 + 2 * (p - 16)], rows_a[par],
                             isa[par])
            pltpu.async_copy(feat_hbm.at[b, sub * 16 + 2 * (p - 16) + 1],
                             rows_b[par], isb[par])

    def wait_in(par):
        pltpu.make_async_copy(x_hbm.at[0, 0], rows_a[par], isa[par]).wait()
        pltpu.make_async_copy(x_hbm.at[0, 0], rows_b[par], isb[par]).wait()

    def start_out(p, par):
        @pl.when(p < 16)
        def _():
            pltpu.async_copy(outs_a[par], fk_hbm.at[b, sub * 32 + 2 * p], osa[par])
            pltpu.async_copy(outs_b[par], fk_hbm.at[b, sub * 32 + 2 * p + 1],
                             osb[par])

        @pl.when(p >= 16)
        def _():
            pltpu.async_copy(outs_a[par], lfk_hbm.at[b, sub * 16 + 2 * (p - 16)],
                             osa[par])
            pltpu.async_copy(outs_b[par], lfk_hbm.at[b, sub * 16 + 2 * (p - 16) + 1],
                             osb[par])

    def wait_out(par):
        pltpu.make_async_copy(outs_a[par], fk_hbm.at[0, 0], osa[par]).wait()
        pltpu.make_async_copy(outs_b[par], fk_hbm.at[0, 0], osb[par]).wait()

    def gather_pair(par, scale):
        @plsc.parallel_loop(0, K // _L, unroll=4)
        def _(i):
            sl = pl.ds(i * _L, _L)
            iv = idx_v[sl]
            ga = plsc.load_gather(rows_a[par], [iv])
            gb = plsc.load_gather(rows_b[par], [iv])
            if scale:
                sc = sck_v[sl]
                ga = ga * sc
                gb = gb * sc
            outs_a[par][sl] = ga
            outs_b[par][sl] = gb

    start_in(0, 0)
    start_in(1, 1)

    def step(p2, _):
        for par in (0, 1):
            p = p2 * 2 + par
            wait_in(par)

            @pl.when(p >= 2)
            def _():
                wait_out(par)

            @pl.when(p < 16)
            def _():
                gather_pair(par, True)

            @pl.when(p >= 16)
            def _():
                gather_pair(par, False)

            start_out(p, par)

            @pl.when(p + 2 < 24)
            def _():
                start_in(p + 2, par)

        return 0

    lax.fori_loop(0, 12, step, 0)
    wait_out(0)
    wait_out(1)

    @pl.when(sub < 3)
    def _():
        pltpu.sync_copy(pos_hbm.at[b, sub], rowa0)

        @plsc.parallel_loop(0, K // _L, unroll=4)
        def _(i):
            sl = pl.ds(i * _L, _L)
            outa0[sl] = plsc.load_gather(rowa0, [idx_v[sl]])

        pltpu.sync_copy(outa0, posk_hbm.at[b, sub])


def _gather(pos, feat, x, idx, sck):
    B, C, N = x.shape
    CF = feat.shape[1]
    K = N // 2
    mesh = plsc.VectorSubcoreMesh(core_axis_name="c", subcore_axis_name="s")
    f = pl.kernel(
        _gather_body,
        out_type=[
            jax.ShapeDtypeStruct((B, 3, K), jnp.float32),
            jax.ShapeDtypeStruct((B, CF, K), jnp.float32),
            jax.ShapeDtypeStruct((B, C, K), jnp.float32),
        ],
        mesh=mesh,
        scratch_types=[
            pltpu.VMEM((K,), jnp.int32),
            pltpu.VMEM((K,), jnp.float32),
            pltpu.VMEM((N,), jnp.float32),
            pltpu.VMEM((N,), jnp.float32),
            pltpu.VMEM((N,), jnp.float32),
            pltpu.VMEM((N,), jnp.float32),
            pltpu.VMEM((K,), jnp.float32),
            pltpu.VMEM((K,), jnp.float32),
            pltpu.VMEM((K,), jnp.float32),
            pltpu.VMEM((K,), jnp.float32),
            pltpu.SemaphoreType.DMA,
            pltpu.SemaphoreType.DMA,
            pltpu.SemaphoreType.DMA,
            pltpu.SemaphoreType.DMA,
            pltpu.SemaphoreType.DMA,
            pltpu.SemaphoreType.DMA,
            pltpu.SemaphoreType.DMA,
            pltpu.SemaphoreType.DMA,
        ],
        compiler_params=pltpu.CompilerParams(needs_layout_passes=False),
    )
    return f(pos, feat, x, idx, sck)


def kernel(pos, feat, x, W1, b1, W2):
    score = _scores(x, W1, b1, W2)
    top_idx, score_k = _sort(score)
    return tuple(_gather(pos, feat, x, top_idx, score_k))


# speedup vs baseline: 1.0590x; 1.0590x over previous
import functools

import jax
import jax.numpy as jnp
from jax import lax
from jax.experimental import pallas as pl
from jax.experimental.pallas import tpu as pltpu
from jax.experimental.pallas import tpu_sc as plsc

_L = 16
_NC = 2
_BITS = 10
_BINS = 1 << _BITS
_KMAX = 0x3FFFFFFF


def _score_body(x_ref, w1_ref, b1_ref, w2_ref, out_ref):
    xb = x_ref[0]
    h = jnp.dot(w1_ref[...], xb, preferred_element_type=jnp.float32)
    h = jax.nn.relu(h + b1_ref[...][:, None])
    s = jnp.dot(w2_ref[...], h, preferred_element_type=jnp.float32)
    out_ref[0] = jax.nn.sigmoid(s)


def _scores(x, W1, b1, W2):
    B, C, N = x.shape
    out = pl.pallas_call(
        _score_body,
        grid=(B,),
        in_specs=[
            pl.BlockSpec((1, C, N), lambda b: (b, 0, 0)),
            pl.BlockSpec((C // 4, C), lambda b: (0, 0)),
            pl.BlockSpec((C // 4,), lambda b: (0,)),
            pl.BlockSpec((1, C // 4), lambda b: (0, 0)),
        ],
        out_specs=pl.BlockSpec((1, 1, N), lambda b: (b, 0, 0)),
        out_shape=jax.ShapeDtypeStruct((B, 1, N), jnp.float32),
    )(x, W1, b1, W2)
    return out[:, 0, :]


def _sort_body(score_hbm, idx_hbm, sck_hbm,
               srow, val0, val1, hist0, hist1, hist2, hist3, tvec, sck_v):
    B, N = score_hbm.shape
    K = N // 2
    vpl = N // _L
    qv = vpl // 4
    wid = lax.axis_index("s") * _NC + lax.axis_index("c")

    @pl.when(wid < B)
    def _():
        lane = lax.iota(jnp.int32, _L)
        pltpu.sync_copy(score_hbm.at[wid], srow)

        lvpl = lane * vpl
        ones = jnp.ones((_L,), jnp.int32)
        zeros = jnp.zeros((_L,), jnp.int32)

        def key_at(vi, iv):
            if vi is None:
                vv = iv
            else:
                vv = plsc.load_gather(vi, [iv])
            u = plsc.bitcast(plsc.load_gather(srow, [vv]), jnp.int32)
            return _KMAX - u, vv

        def rs_pass(vi, vo, shift):
            @plsc.parallel_loop(0, _BINS, unroll=4)
            def _(i):
                sl = pl.ds(i * _L, _L)
                hist0[sl] = zeros
                hist1[sl] = zeros
                hist2[sl] = zeros
                hist3[sl] = zeros

            def h1(v, _):
                k0, _v0 = key_at(vi, lvpl + v)
                k1, _v1 = key_at(vi, lvpl + (qv + v))
                k2, _v2 = key_at(vi, lvpl + (2 * qv + v))
                k3, _v3 = key_at(vi, lvpl + (3 * qv + v))
                a0 = ((k0 >> shift) & (_BINS - 1)) * _L + lane
                a1 = ((k1 >> shift) & (_BINS - 1)) * _L + lane
                a2 = ((k2 >> shift) & (_BINS - 1)) * _L + lane
                a3 = ((k3 >> shift) & (_BINS - 1)) * _L + lane
                plsc.addupdate_scatter(hist0, [a0], ones)
                plsc.addupdate_scatter(hist1, [a1], ones)
                plsc.addupdate_scatter(hist2, [a2], ones)
                plsc.addupdate_scatter(hist3, [a3], ones)
                return 0

            lax.fori_loop(0, qv, h1, 0, unroll=2)

            @plsc.parallel_loop(0, _BINS, unroll=4)
            def _(i):
                sl = pl.ds(i * _L, _L)
                c0 = hist0[sl]
                c1 = hist1[sl]
                c2 = hist2[sl]
                c3 = hist3[sl]
                t = c0 + c1 + c2 + c3
                ic = plsc.cumsum(t)
                r0 = ic - t
                hist0[sl] = r0
                hist1[sl] = r0 + c0
                hist2[sl] = r0 + c0 + c1
                hist3[sl] = ic

            def lvl2(i, carry):
                addr15 = (i * _L + lane) * _L + (_L - 1)
                t = plsc.load_gather(hist3, [addr15])
                eb = (plsc.cumsum(t) - t) + carry
                plsc.store_scatter(tvec, [i * _L + lane], eb)
                return carry + jnp.sum(t)

            lax.fori_loop(0, _BINS // _L, lvl2, jnp.int32(0))

            @plsc.parallel_loop(0, _BINS, unroll=4)
            def _(i):
                sl = pl.ds(i * _L, _L)
                base = plsc.load_gather(tvec, [lane * 0 + i])
                hist0[sl] = hist0[sl] + base
                hist1[sl] = hist1[sl] + base
                hist2[sl] = hist2[sl] + base
                hist3[sl] = hist3[sl] + base

            def h3(v, _):
                k0, v0 = key_at(vi, lvpl + v)
                k1, v1 = key_at(vi, lvpl + (qv + v))
                k2, v2 = key_at(vi, lvpl + (2 * qv + v))
                k3, v3 = key_at(vi, lvpl + (3 * qv + (qv - 1 - v)))
                a0 = ((k0 >> shift) & (_BINS - 1)) * _L + lane
                a1 = ((k1 >> shift) & (_BINS - 1)) * _L + lane
                a2 = ((k2 >> shift) & (_BINS - 1)) * _L + lane
                a3 = ((k3 >> shift) & (_BINS - 1)) * _L + lane
                cur0 = plsc.load_gather(hist0, [a0])
                cur1 = plsc.load_gather(hist1, [a1])
                cur2 = plsc.load_gather(hist2, [a2])
                cur3 = plsc.load_gather(hist3, [a3]) - 1
                plsc.store_scatter(hist0, [a0], cur0 + 1)
                plsc.store_scatter(hist1, [a1], cur1 + 1)
                plsc.store_scatter(hist2, [a2], cur2 + 1)
                plsc.store_scatter(hist3, [a3], cur3)
                plsc.store_scatter(vo, [cur0], v0)
                plsc.store_scatter(vo, [cur1], v1)
                plsc.store_scatter(vo, [cur2], v2)
                plsc.store_scatter(vo, [cur3], v3)
                return 0

            lax.fori_loop(0, qv, h3, 0, unroll=2)

        rs_pass(None, val1, 0)
        rs_pass(val1, val0, _BITS)
        rs_pass(val0, val1, 2 * _BITS)

        @plsc.parallel_loop(0, K // _L, unroll=4)
        def _(i):
            sl = pl.ds(i * _L, _L)
            sck_v[sl] = plsc.load_gather(srow, [val1[sl]])

        pltpu.sync_copy(val1.at[pl.ds(0, K)], idx_hbm.at[wid])
        pltpu.sync_copy(sck_v, sck_hbm.at[wid])


def _sort(score):
    B, N = score.shape
    K = N // 2
    mesh = plsc.VectorSubcoreMesh(core_axis_name="c", subcore_axis_name="s")
    f = pl.kernel(
        _sort_body,
        out_type=[
            jax.ShapeDtypeStruct((B, K), jnp.int32),
            jax.ShapeDtypeStruct((B, K), jnp.float32),
        ],
        mesh=mesh,
        scratch_types=[
            pltpu.VMEM((N,), jnp.float32),
            pltpu.VMEM((N,), jnp.int32),
            pltpu.VMEM((N,), jnp.int32),
            pltpu.VMEM((_BINS * _L,), jnp.int32),
            pltpu.VMEM((_BINS * _L,), jnp.int32),
            pltpu.VMEM((_BINS * _L,), jnp.int32),
            pltpu.VMEM((_BINS * _L,), jnp.int32),
            pltpu.VMEM((_BINS,), jnp.int32),
            pltpu.VMEM((K,), jnp.float32),
        ],
        compiler_params=pltpu.CompilerParams(needs_layout_passes=False),
    )
    return f(score)


def _gather_body(pos_hbm, feat_hbm, x_hbm, idx_hbm, sck_hbm,
                 posk_hbm, lfk_hbm, fk_hbm,
                 idx_v, sck_v, rowa0, rowb0, rowa1, rowb1,
                 outa0, outb0, outa1, outb1,
                 isa0, isb0, isa1, isb1, osa0, osb0, osa1, osb1):
    B = pos_hbm.shape[0]
    C = x_hbm.shape[1]
    CF = feat_hbm.shape[1]
    N = x_hbm.shape[2]
    K = N // 2
    wid = lax.axis_index("s") * _NC + lax.axis_index("c")
    b = wid // 4
    sub = wid % 4
    rows_a = (rowa0, rowa1)
    rows_b = (rowb0, rowb1)
    outs_a = (outa0, outa1)
    outs_b = (outb0, outb1)
    isa = (isa0, isa1)
    isb = (isb0, isb1)
    osa = (osa0, osa1)
    osb = (osb0, osb1)

    pltpu.sync_copy(idx_hbm.at[b], idx_v)
    pltpu.sync_copy(sck_hbm.at[b], sck_v)

    def start_in(p, par):
        @pl.when(p < 16)
        def _():
            pltpu.async_copy(x_hbm.at[b, sub * 32 + 2 * p], rows_a[par], isa[par])
            pltpu.async_copy(x_hbm.at[b, sub * 32 + 2 * p + 1], rows_b[par],
                             isb[par])

        @pl.when(p >= 16)
        def _():
            pltpu.async_copy(feat_hbm.at[b, sub * 16 + 2 * (p - 16)], rows_a[par],
                             isa[par])
            pltpu.async_copy(feat_hbm.at[b, sub * 16 + 2 * (p - 16) + 1],
                             rows_b[par], isb[par])

    def wait_in(par):
        pltpu.make_async_copy(x_hbm.at[0, 0], rows_a[par], isa[par]).wait()
        pltpu.make_async_copy(x_hbm.at[0, 0], rows_b[par], isb[par]).wait()

    def start_out(p, par):
        @pl.when(p < 16)
        def _():
            pltpu.async_copy(outs_a[par], fk_hbm.at[b, sub * 32 + 2 * p], osa[par])
            pltpu.async_copy(outs_b[par], fk_hbm.at[b, sub * 32 + 2 * p + 1],
                             osb[par])

        @pl.when(p >= 16)
        def _():
            pltpu.async_copy(outs_a[par], lfk_hbm.at[b, sub * 16 + 2 * (p - 16)],
                             osa[par])
            pltpu.async_copy(outs_b[par], lfk_hbm.at[b, sub * 16 + 2 * (p - 16) + 1],
                             osb[par])

    def wait_out(par):
        pltpu.make_async_copy(outs_a[par], fk_hbm.at[0, 0], osa[par]).wait()
        pltpu.make_async_copy(outs_b[par], fk_hbm.at[0, 0], osb[par]).wait()

    def gather_pair(par, scale):
        @plsc.parallel_loop(0, K // _L, unroll=4)
        def _(i):
            sl = pl.ds(i * _L, _L)
            iv = idx_v[sl]
            ga = plsc.load_gather(rows_a[par], [iv])
            gb = plsc.load_gather(rows_b[par], [iv])
            if scale:
                sc = sck_v[sl]
                ga = ga * sc
                gb = gb * sc
            outs_a[par][sl] = ga
            outs_b[par][sl] = gb

    start_in(0, 0)
    start_in(1, 1)

    def step(p2, _):
        for par in (0, 1):
            p = p2 * 2 + par
            wait_in(par)

            @pl.when(p >= 2)
            def _():
                wait_out(par)

            @pl.when(p < 16)
            def _():
                gather_pair(par, True)

            @pl.when(p >= 16)
            def _():
                gather_pair(par, False)

            start_out(p, par)

            @pl.when(p + 2 < 24)
            def _():
                start_in(p + 2, par)

        return 0

    lax.fori_loop(0, 12, step, 0)
    wait_out(0)
    wait_out(1)

    @pl.when(sub < 3)
    def _():
        pltpu.sync_copy(pos_hbm.at[b, sub], rowa0)

        @plsc.parallel_loop(0, K // _L, unroll=4)
        def _(i):
            sl = pl.ds(i * _L, _L)
            outa0[sl] = plsc.load_gather(rowa0, [idx_v[sl]])

        pltpu.sync_copy(outa0, posk_hbm.at[b, sub])


def _gather(pos, feat, x, idx, sck):
    B, C, N = x.shape
    CF = feat.shape[1]
    K = N // 2
    mesh = plsc.VectorSubcoreMesh(core_axis_name="c", subcore_axis_name="s")
    f = pl.kernel(
        _gather_body,
        out_type=[
            jax.ShapeDtypeStruct((B, 3, K), jnp.float32),
            jax.ShapeDtypeStruct((B, CF, K), jnp.float32),
            jax.ShapeDtypeStruct((B, C, K), jnp.float32),
        ],
        mesh=mesh,
        scratch_types=[
            pltpu.VMEM((K,), jnp.int32),
            pltpu.VMEM((K,), jnp.float32),
            pltpu.VMEM((N,), jnp.float32),
            pltpu.VMEM((N,), jnp.float32),
            pltpu.VMEM((N,), jnp.float32),
            pltpu.VMEM((N,), jnp.float32),
            pltpu.VMEM((K,), jnp.float32),
            pltpu.VMEM((K,), jnp.float32),
            pltpu.VMEM((K,), jnp.float32),
            pltpu.VMEM((K,), jnp.float32),
            pltpu.SemaphoreType.DMA,
            pltpu.SemaphoreType.DMA,
            pltpu.SemaphoreType.DMA,
            pltpu.SemaphoreType.DMA,
            pltpu.SemaphoreType.DMA,
            pltpu.SemaphoreType.DMA,
            pltpu.SemaphoreType.DMA,
            pltpu.SemaphoreType.DMA,
        ],
        compiler_params=pltpu.CompilerParams(needs_layout_passes=False),
    )
    return f(pos, feat, x, idx, sck)


def kernel(pos, feat, x, W1, b1, W2):
    score = _scores(x, W1, b1, W2)
    top_idx, score_k = _sort(score)
    return tuple(_gather(pos, feat, x, top_idx, score_k))
